# Initial kernel scaffold; baseline (speedup 1.0000x reference)
import jax
import jax.numpy as jnp
from jax.experimental import pallas as pl


def kernel(prediction, target, mask):
    raise NotImplementedError("write your pallas kernel here")


# speedup vs baseline: 18.0500x; 18.0500x over previous
import functools

import jax
import jax.numpy as jnp
from jax import lax
from jax.experimental import pallas as pl
from jax.experimental.pallas import tpu as pltpu
from jax.experimental.pallas import tpu_sc as plsc

ALPHA = 0.5
TRIM = 0.2
NSCALES = 4
B, H, W = 32, 512, 512
N = H * W
NB = 8192
CHUNK = 4096


def _moments_body(p_ref, t_ref, m_ref, out_ref):
    p = p_ref[0]
    t = t_ref[0]
    m = m_ref[0].astype(jnp.float32)
    mp = m * p
    a00 = jnp.sum(mp * p)
    a01 = jnp.sum(mp)
    a11 = jnp.sum(m)
    b0 = jnp.sum(mp * t)
    b1 = jnp.sum(m * t)
    lane = lax.broadcasted_iota(jnp.int32, (1, 128), 1)
    out = jnp.where(lane == 0, a00, 0.0)
    out = jnp.where(lane == 1, a01, out)
    out = jnp.where(lane == 2, a11, out)
    out = jnp.where(lane == 3, b0, out)
    out = jnp.where(lane == 4, b1, out)
    out_ref[0] = out


def _moments(prediction, target, mask):
    return pl.pallas_call(
        _moments_body,
        grid=(B,),
        in_specs=[
            pl.BlockSpec((1, H, W), lambda i: (i, 0, 0)),
            pl.BlockSpec((1, H, W), lambda i: (i, 0, 0)),
            pl.BlockSpec((1, H, W), lambda i: (i, 0, 0)),
        ],
        out_specs=pl.BlockSpec((1, 1, 128), lambda i: (i, 0, 0)),
        out_shape=jax.ShapeDtypeStruct((B, 1, 128), jnp.float32),
    )(prediction, target, mask)


def _grad_body(s_ref, p_ref, t_ref, m_ref, out_ref):
    s = s_ref[0, 0, 0]
    p = p_ref[0]
    t = t_ref[0]
    m = m_ref[0].astype(jnp.float32)
    row = lax.broadcasted_iota(jnp.int32, (H, W), 0)
    col = lax.broadcasted_iota(jnp.int32, (H, W), 1)
    lane = lax.broadcasted_iota(jnp.int32, (1, 128), 1)
    out = jnp.zeros((1, 128), jnp.float32)
    for k, st in enumerate((1, 2, 4, 8)):
        wr = ((row & (st - 1)) == 0).astype(jnp.float32)
        wc = ((col & (st - 1)) == 0).astype(jnp.float32)
        wgt = wr * wc
        mmx = m[:, st:] * m[:, : W - st] * wgt[:, : W - st]
        gx = jnp.sum(
            jnp.abs(s * (p[:, st:] - p[:, : W - st]) - (t[:, st:] - t[:, : W - st]))
            * mmx
        )
        mmy = m[st:, :] * m[: H - st, :] * wgt[: H - st, :]
        gy = jnp.sum(
            jnp.abs(s * (p[st:, :] - p[: H - st, :]) - (t[st:, :] - t[: H - st, :]))
            * mmy
        )
        cnt = jnp.sum(m * wgt)
        out = jnp.where(lane == k, gx + gy, out)
        out = jnp.where(lane == 4 + k, cnt, out)
    out_ref[0] = out


def _gradients(scale, prediction, target, mask):
    return pl.pallas_call(
        _grad_body,
        grid=(B,),
        in_specs=[
            pl.BlockSpec((1, 1, 1), lambda i: (i, 0, 0), memory_space=pltpu.SMEM),
            pl.BlockSpec((1, H, W), lambda i: (i, 0, 0)),
            pl.BlockSpec((1, H, W), lambda i: (i, 0, 0)),
            pl.BlockSpec((1, H, W), lambda i: (i, 0, 0)),
        ],
        out_specs=pl.BlockSpec((1, 1, 128), lambda i: (i, 0, 0)),
        out_shape=jax.ShapeDtypeStruct((B, 1, 128), jnp.float32),
    )(scale.reshape(B, 1, 1), prediction, target, mask)


def _sc_select_body(p_hbm, t_hbm, m_hbm, prm_hbm, out_hbm,
                    prm_v, p_v, t_v, m_v, cnt_v, sum_v, o_v):
    wid = lax.axis_index("s") * 2 + lax.axis_index("c")
    pltpu.sync_copy(prm_hbm.at[wid], prm_v)
    prm = prm_v[...]
    scale = prm[0]
    shift = prm[1]
    inv_bw = prm[2]
    kf = prm[3]

    zeros16 = jnp.zeros((16,), jnp.float32)
    ones16 = jnp.ones((16,), jnp.float32)

    def zinit(i, c):
        cnt_v[pl.ds(i * 16, 16)] = zeros16
        sum_v[pl.ds(i * 16, 16)] = zeros16
        return c

    lax.fori_loop(0, NB // 16, zinit, 0)

    def chunk_body(ci, c):
        off = ci * CHUNK
        pltpu.sync_copy(p_hbm.at[wid, pl.ds(off, CHUNK)], p_v)
        pltpu.sync_copy(t_hbm.at[wid, pl.ds(off, CHUNK)], t_v)
        pltpu.sync_copy(m_hbm.at[wid, pl.ds(off, CHUNK)], m_v)

        def vec_body(i, cc):
            sl = pl.ds(i * 16, 16)
            res = jnp.abs(scale * p_v[sl] + shift - t_v[sl])
            bin_ = jnp.minimum((res * inv_bw).astype(jnp.int32), NB - 1)
            msk = m_v[sl] > 0
            plsc.addupdate_scatter(cnt_v, [bin_], ones16, mask=msk)
            plsc.addupdate_scatter(sum_v, [bin_], res, mask=msk)
            return cc

        lax.fori_loop(0, CHUNK // 16, vec_body, c)
        return c

    lax.fori_loop(0, N // CHUNK, chunk_body, 0)

    def sel_body(i, carry):
        cacc, sacc = carry
        sl = pl.ds(i * 16, 16)
        c = cnt_v[sl]
        s = sum_v[sl]
        excl = plsc.cumsum(c) - c
        take = jnp.clip(kf - cacc - excl, 0.0, c)
        sacc = sacc + jnp.sum(take * (s / jnp.maximum(c, 1.0)))
        cacc = cacc + jnp.sum(c)
        return cacc, sacc

    _, kept = lax.fori_loop(0, NB // 16, sel_body, (0.0, 0.0))

    lanes = lax.iota(jnp.int32, 16)
    o_v[...] = jnp.where(lanes == 0, kept, 0.0)
    pltpu.sync_copy(o_v, out_hbm.at[wid])


@functools.lru_cache(maxsize=1)
def _get_sc_select():
    mesh = plsc.VectorSubcoreMesh(core_axis_name="c", subcore_axis_name="s")

    @functools.partial(
        pl.kernel,
        out_type=jax.ShapeDtypeStruct((B, 16), jnp.float32),
        mesh=mesh,
        scratch_types=[
            pltpu.VMEM((16,), jnp.float32),
            pltpu.VMEM((CHUNK,), jnp.float32),
            pltpu.VMEM((CHUNK,), jnp.float32),
            pltpu.VMEM((CHUNK,), jnp.int32),
            pltpu.VMEM((NB,), jnp.float32),
            pltpu.VMEM((NB,), jnp.float32),
            pltpu.VMEM((16,), jnp.float32),
        ],
        compiler_params=pltpu.CompilerParams(needs_layout_passes=False),
    )
    def _sc_select(*refs):
        _sc_select_body(*refs)

    return _sc_select


def kernel(prediction, target, mask):
    mom = _moments(prediction, target, mask)[:, 0]
    a00, a01, a11 = mom[:, 0], mom[:, 1], mom[:, 2]
    b0, b1 = mom[:, 3], mom[:, 4]
    det = a00 * a11 - a01 * a01
    safe = jnp.where(det != 0, det, 1.0)
    scale = jnp.where(det != 0, (a11 * b0 - a01 * b1) / safe, 0.0)
    shift = jnp.where(det != 0, (-a01 * b0 + a00 * b1) / safe, 0.0)

    m_count = a11
    num_keep = jnp.floor(m_count * (1.0 - TRIM))
    divisor = jnp.sum(m_count * (1.0 - TRIM))

    c0 = jnp.abs(shift)
    c1 = jnp.abs(shift - 1.0)
    c2 = jnp.abs(scale + shift)
    c3 = jnp.abs(scale + shift - 1.0)
    rmax = jnp.maximum(jnp.maximum(c0, c1), jnp.maximum(c2, c3))
    inv_bw = jnp.where(rmax > 0, NB / rmax, 1.0)

    params = jnp.zeros((B, 16), jnp.float32)
    params = params.at[:, 0].set(scale)
    params = params.at[:, 1].set(shift)
    params = params.at[:, 2].set(inv_bw)
    params = params.at[:, 3].set(num_keep)

    grad = _gradients(scale, prediction, target, mask)[:, 0]

    pf = prediction.reshape(B, N)
    tf = target.reshape(B, N)
    mf = mask.reshape(B, N)
    sel = _get_sc_select()(pf, tf, mf, params)
    image_loss = sel[:, 0]
    safe_div = jnp.where(divisor == 0, 1.0, divisor)
    data_loss = jnp.where(divisor == 0, 0.0, jnp.sum(image_loss) / safe_div)

    reg_loss = 0.0
    for k in range(NSCALES):
        g = jnp.sum(grad[:, k])
        c = jnp.sum(grad[:, 4 + k])
        safe_c = jnp.where(c == 0, 1.0, c)
        reg_loss = reg_loss + jnp.where(c == 0, 0.0, g / safe_c)

    total = data_loss + ALPHA * reg_loss
    return total, data_loss, reg_loss
